# baseline (speedup 1.0000x reference)
import functools

import jax
import jax.numpy as jnp
from jax import lax
from jax.experimental import pallas as pl
from jax.experimental.pallas import tpu as pltpu

E = 8
K = 2
B = 32
D = 4096
F = 2048

DB = 512
FB = 512
ND = D // DB
NF = F // FB
NT = ND + NF


def _routing_weights(x, gate_w):
    logits = jnp.dot(x, gate_w, preferred_element_type=jnp.float32)
    idx = lax.broadcasted_iota(jnp.int32, (B, E), 1)
    m1 = jnp.max(logits, axis=1, keepdims=True)
    i1 = jnp.min(jnp.where(logits == m1, idx, E), axis=1, keepdims=True)
    masked = jnp.where(idx == i1, -jnp.inf, logits)
    m2 = jnp.max(masked, axis=1, keepdims=True)
    i2 = jnp.min(jnp.where(masked == m2, idx, E), axis=1, keepdims=True)
    t = jnp.exp(m2 - m1)
    p1 = 1.0 / (1.0 + t)
    p2 = t / (1.0 + t)
    return jnp.where(idx == i1, p1, 0.0) + jnp.where(idx == i2, p2, 0.0)


def _moe_body(x_ref, gw_ref, w1_ref, w3_ref, w2_ref, out_ref,
              we_ref, a1_ref, a3_ref, h_ref):
    e = pl.program_id(0)
    t = pl.program_id(1)

    @pl.when((e == 0) & (t == 0))
    def _init():
        we_ref[...] = _routing_weights(x_ref[...], gw_ref[...])
        out_ref[...] = jnp.zeros_like(out_ref)

    @pl.when(t == 0)
    def _zero_acc():
        a1_ref[...] = jnp.zeros_like(a1_ref)
        a3_ref[...] = jnp.zeros_like(a3_ref)

    @pl.when(t < ND)
    def _phase1():
        xb = x_ref[:, pl.ds(t * DB, DB)].astype(jnp.bfloat16)
        a1_ref[...] += jnp.dot(xb, w1_ref[0].astype(jnp.bfloat16),
                               preferred_element_type=jnp.float32)
        a3_ref[...] += jnp.dot(xb, w3_ref[0].astype(jnp.bfloat16),
                               preferred_element_type=jnp.float32)

    @pl.when(t == ND)
    def _activate():
        a1 = a1_ref[...]
        eidx = lax.broadcasted_iota(jnp.int32, (B, E), 1)
        wcol = jnp.sum(jnp.where(eidx == e, we_ref[...], 0.0), axis=1,
                       keepdims=True)
        h = (a1 / (1.0 + jnp.exp(-a1))) * a3_ref[...] * wcol
        h_ref[...] = h.astype(jnp.bfloat16)

    @pl.when(t >= ND)
    def _phase2():
        fb = t - ND
        out_ref[...] += jnp.dot(h_ref[:, pl.ds(fb * FB, FB)],
                                w2_ref[0].astype(jnp.bfloat16),
                                preferred_element_type=jnp.float32)


@jax.jit
def kernel(x, gate_w, w1, w3, w2):
    grid = (E, NT)
    return pl.pallas_call(
        _moe_body,
        grid=grid,
        in_specs=[
            pl.BlockSpec((B, D), lambda e, t: (0, 0)),
            pl.BlockSpec((D, E), lambda e, t: (0, 0)),
            pl.BlockSpec((1, DB, F), lambda e, t: (e, jnp.minimum(t, ND - 1), 0)),
            pl.BlockSpec((1, DB, F), lambda e, t: (e, jnp.minimum(t, ND - 1), 0)),
            pl.BlockSpec((1, FB, D), lambda e, t: (e, jnp.maximum(t - ND, 0), 0)),
        ],
        out_specs=pl.BlockSpec((B, D), lambda e, t: (0, 0)),
        out_shape=jax.ShapeDtypeStruct((B, D), jnp.float32),
        scratch_shapes=[
            pltpu.VMEM((B, E), jnp.float32),
            pltpu.VMEM((B, F), jnp.float32),
            pltpu.VMEM((B, F), jnp.float32),
            pltpu.VMEM((B, F), jnp.bfloat16),
        ],
        compiler_params=pltpu.CompilerParams(
            dimension_semantics=("arbitrary", "arbitrary"),
        ),
    )(x, gate_w, w1, w3, w2)


# speedup vs baseline: 1.0725x; 1.0725x over previous
import functools

import jax
import jax.numpy as jnp
from jax import lax
from jax.experimental import pallas as pl
from jax.experimental.pallas import tpu as pltpu

E = 8
K = 2
B = 32
D = 4096
F = 2048

FB = 256
NF = F // FB


def _routing_weights(x, gate_w):
    logits = jnp.dot(x, gate_w, preferred_element_type=jnp.float32)
    idx = lax.broadcasted_iota(jnp.int32, (B, E), 1)
    m1 = jnp.max(logits, axis=1, keepdims=True)
    i1 = jnp.min(jnp.where(logits == m1, idx, E), axis=1, keepdims=True)
    masked = jnp.where(idx == i1, -jnp.inf, logits)
    m2 = jnp.max(masked, axis=1, keepdims=True)
    i2 = jnp.min(jnp.where(masked == m2, idx, E), axis=1, keepdims=True)
    t = jnp.exp(m2 - m1)
    p1 = 1.0 / (1.0 + t)
    p2 = t / (1.0 + t)
    return jnp.where(idx == i1, p1, 0.0) + jnp.where(idx == i2, p2, 0.0)


def _moe_body(x_ref, gw_ref, w1_ref, w3_ref, w2_ref, out_ref, we_ref, acc_ref):
    e = pl.program_id(0)
    f = pl.program_id(1)

    @pl.when((e == 0) & (f == 0))
    def _init():
        we_ref[...] = _routing_weights(x_ref[...], gw_ref[...])
        out_ref[...] = jnp.zeros_like(out_ref)

    @pl.when(f == 0)
    def _zero_acc():
        acc_ref[...] = jnp.zeros_like(acc_ref)

    x = x_ref[...].astype(jnp.bfloat16)
    w1b = w1_ref[0].astype(jnp.bfloat16)
    w3b = w3_ref[0].astype(jnp.bfloat16)
    a1 = jnp.dot(x, w1b, preferred_element_type=jnp.float32)
    a3 = jnp.dot(x, w3b, preferred_element_type=jnp.float32)
    h = (a1 / (1.0 + jnp.exp(-a1))) * a3
    acc_ref[...] += jnp.dot(h.astype(jnp.bfloat16),
                            w2_ref[0].astype(jnp.bfloat16),
                            preferred_element_type=jnp.float32)

    @pl.when(f == NF - 1)
    def _combine():
        eidx = lax.broadcasted_iota(jnp.int32, (B, E), 1)
        wcol = jnp.sum(jnp.where(eidx == e, we_ref[...], 0.0), axis=1,
                       keepdims=True)
        out_ref[...] += acc_ref[...] * wcol


@jax.jit
def kernel(x, gate_w, w1, w3, w2):
    grid = (E, NF)
    return pl.pallas_call(
        _moe_body,
        grid=grid,
        in_specs=[
            pl.BlockSpec((B, D), lambda e, f: (0, 0)),
            pl.BlockSpec((D, E), lambda e, f: (0, 0)),
            pl.BlockSpec((1, D, FB), lambda e, f: (e, 0, f)),
            pl.BlockSpec((1, D, FB), lambda e, f: (e, 0, f)),
            pl.BlockSpec((1, FB, D), lambda e, f: (e, f, 0)),
        ],
        out_specs=pl.BlockSpec((B, D), lambda e, f: (0, 0)),
        out_shape=jax.ShapeDtypeStruct((B, D), jnp.float32),
        scratch_shapes=[
            pltpu.VMEM((B, E), jnp.float32),
            pltpu.VMEM((B, D), jnp.float32),
        ],
        compiler_params=pltpu.CompilerParams(
            dimension_semantics=("arbitrary", "arbitrary"),
        ),
    )(x, gate_w, w1, w3, w2)
